# baseline (speedup 1.0000x reference)
import functools

import jax
import jax.numpy as jnp
from jax import lax
from jax.experimental import pallas as pl
from jax.experimental.pallas import tpu as pltpu
from jax.experimental.pallas import tpu_sc as plsc

_N = 10000
_E = 160000
_IN = 256
_HID = 512
_OUT = 256
_D = _OUT
_HOPS = 10
_ALPHA = 0.1

_NC = 2
_NS = 16
_L = 16
_NW = _NC * _NS
_NP = 2
_QR = 2560
_RPT = _QR // _NS
_NPAD = _NC * _NP * _QR
_CAPE = 4096
_CHUNK = 64
_EPAD = _E + _CAPE


def _mlp_body(x_ref, w1_ref, b1_ref, w2_ref, b2_ref, o_ref, oa_ref):
    h = jnp.dot(x_ref[...], w1_ref[...], preferred_element_type=jnp.float32)
    h = jnp.maximum(h + b1_ref[...], 0.0)
    o = jnp.dot(h, w2_ref[...], preferred_element_type=jnp.float32)
    o = o + b2_ref[...]
    o_ref[...] = o
    oa_ref[...] = o * _ALPHA


def _mlp(X, W1, b1, W2, b2):
    BR = 1000
    return pl.pallas_call(
        _mlp_body,
        grid=(_N // BR,),
        in_specs=[
            pl.BlockSpec((BR, _IN), lambda i: (i, 0)),
            pl.BlockSpec((_IN, _HID), lambda i: (0, 0)),
            pl.BlockSpec((1, _HID), lambda i: (0, 0)),
            pl.BlockSpec((_HID, _OUT), lambda i: (0, 0)),
            pl.BlockSpec((1, _OUT), lambda i: (0, 0)),
        ],
        out_specs=[
            pl.BlockSpec((BR, _OUT), lambda i: (i, 0)),
            pl.BlockSpec((BR, _OUT), lambda i: (i, 0)),
        ],
        out_shape=[
            jax.ShapeDtypeStruct((_N, _OUT), jnp.float32),
            jax.ShapeDtypeStruct((_N, _OUT), jnp.float32),
        ],
    )(X, W1, b1.reshape(1, _HID), W2, b2.reshape(1, _OUT))


def _splat(v, j):
    idx = jnp.full((_L, 1), j, jnp.int32)
    dn = lax.GatherDimensionNumbers(
        offset_dims=(), collapsed_slice_dims=(0,), start_index_map=(0,))
    return lax.gather(v, idx, dn, slice_sizes=(1,),
                      mode=lax.GatherScatterMode.PROMISE_IN_BOUNDS)


def _make_hop():
    mesh = plsc.VectorSubcoreMesh(
        core_axis_name="c", subcore_axis_name="s",
        num_cores=_NC, num_subcores=_NS)

    @functools.partial(
        pl.kernel,
        out_type=jax.ShapeDtypeStruct((_NPAD, _D), jnp.float32),
        mesh=mesh,
        scratch_types=[
            pltpu.VMEM((_CAPE,), jnp.int32),
            pltpu.VMEM((_CAPE,), jnp.float32),
            pltpu.VMEM((_CAPE // _CHUNK, _CHUNK), jnp.int32),
            pltpu.VMEM((_CHUNK, _D), jnp.float32),
            pltpu.VMEM((_CHUNK, _D), jnp.float32),
            pltpu.VMEM((_CHUNK, _D), jnp.float32),
            pltpu.VMEM((_CHUNK, _D), jnp.float32),
            pltpu.VMEM((1056,), jnp.int32),
            pltpu.VMEM_SHARED((_QR, _D), jnp.float32),
            pltpu.SemaphoreType.DMA,
            pltpu.SemaphoreType.DMA,
            pltpu.SemaphoreType.DMA,
            pltpu.SemaphoreType.DMA,
        ],
        compiler_params=pltpu.CompilerParams(
            use_tc_tiling_on_sc=False, needs_layout_passes=False),
    )
    def hop(z_hbm, z0a_hbm, col_hbm, val_hbm, idx_hbm, st_hbm, out_hbm,
            colseg, valseg, idxseg, rb0, rb1, sb0, sb1, stv, accs,
            g0, g1, t0, t1):
        cid = lax.axis_index("c")
        sid = lax.axis_index("s")
        wid = sid * _NC + cid
        srow = sid * _RPT
        iota = lax.iota(jnp.int32, _L)

        pltpu.sync_copy(st_hbm, stv)

        def _gather(ci, rb, sem):
            src = z_hbm.at[colseg.at[pl.ds(ci * _CHUNK, _CHUNK)]]
            pltpu.async_copy(src, rb, sem)

        def _gwait(ci, rb, sem):
            src = z_hbm.at[colseg.at[pl.ds(ci * _CHUNK, _CHUNK)]]
            pltpu.make_async_copy(src, rb, sem).wait()

        def _scatter(ci, sb, sem):
            pltpu.async_copy(sb, accs.at[idxseg.at[ci]], sem, add=True)

        def _twait(sb, sem):
            pltpu.make_async_copy(sb, accs.at[idxseg.at[0]], sem).wait()

        def _scale(cs, ci, rb, sb, start, end):
            ebase = ci * _CHUNK
            nk = _D // _L
            lat = 4
            for g in range(_CHUNK // _L):
                off = ebase + g * _L
                eidx = (cs + off) + iota
                valid = (eidx >= start) & (eidx < end)
                valv = valseg[pl.ds(off, _L)]
                a = jnp.where(valid, valv, 0.0)
                for j in range(_L):
                    av = _splat(a, j)
                    rref = rb.at[g * _L + j]
                    sref = sb.at[g * _L + j]
                    rv = [None] * nk
                    pv = [None] * nk
                    for t in range(nk + lat + 1):
                        if t < nk:
                            rv[t] = rref[pl.ds(t * _L, _L)]
                        if lat <= t < nk + lat:
                            k = t - lat
                            pv[k] = av * rv[k]
                        if lat + 1 <= t:
                            k = t - lat - 1
                            sref[pl.ds(k * _L, _L)] = pv[k]

        for p in range(_NP):
            tb = wid * (_NP * 16) + p * 16
            start = stv[pl.ds(pl.multiple_of(tb, 8), _L)][0]
            end = stv[pl.ds(pl.multiple_of(tb + 8, 8), _L)][0]
            grow = (cid * _NP + p) * _QR + sid * _RPT

            pltpu.sync_copy(z0a_hbm.at[pl.ds(grow, _RPT)],
                            accs.at[pl.ds(srow, _RPT)])
            plsc.subcore_barrier()

            astart = start & (-256)
            total = end - astart
            nseg = lax.div(total + (_CAPE - 1), _CAPE)

            @pl.loop(0, nseg)
            def _seg(si):
                soff = pl.multiple_of(astart + si * _CAPE, 256)
                pltpu.sync_copy(col_hbm.at[pl.ds(soff, _CAPE)], colseg)
                pltpu.sync_copy(val_hbm.at[pl.ds(soff, _CAPE)], valseg)
                pltpu.sync_copy(
                    idx_hbm.at[pl.ds(lax.div(soff, _CHUNK), _CAPE // _CHUNK)],
                    idxseg)
                seg_n = jnp.minimum(end - soff, _CAPE)
                nch = lax.div(seg_n + (_CHUNK - 1), _CHUNK)
                nch2 = lax.div(nch + 1, 2) * 2

                _gather(0, rb0, g0)
                _gather(1, rb1, g1)

                @pl.loop(0, nch2, step=2)
                def _c(ci):
                    _gwait(ci, rb0, g0)

                    @pl.when(ci > 0)
                    def _():
                        _twait(sb0, t0)

                    _scale(soff, ci, rb0, sb0, start, end)
                    _scatter(ci, sb0, t0)

                    @pl.when(ci + 2 < nch2)
                    def _():
                        _gather(ci + 2, rb0, g0)

                    _gwait(ci + 1, rb1, g1)

                    @pl.when(ci > 0)
                    def _():
                        _twait(sb1, t1)

                    _scale(soff, ci + 1, rb1, sb1, start, end)
                    _scatter(ci + 1, sb1, t1)

                    @pl.when(ci + 3 < nch2)
                    def _():
                        _gather(ci + 3, rb1, g1)

                _twait(sb0, t0)
                _twait(sb1, t1)

            plsc.subcore_barrier()
            pltpu.sync_copy(accs.at[pl.ds(srow, _RPT)],
                            out_hbm.at[pl.ds(grow, _RPT)])

    return hop


_hop = _make_hop()


def kernel(X, edge_index, A_val, W1, b1, W2, b2):
    Z0, Z0a = _mlp(X, W1, b1, W2, b2)

    row = edge_index[0]
    col = edge_index[1]
    order = jnp.argsort(row)
    sdst = row[order]
    scol = col[order]
    sval = A_val[order] * (1.0 - _ALPHA)

    qb = jnp.searchsorted(
        sdst, jnp.arange(_NC * _NP + 1, dtype=jnp.int32) * _QR, side="left"
    ).astype(jnp.int32)
    nq = _NW * _NP
    k = jnp.arange(nq, dtype=jnp.int32)
    wids = k // _NP
    pw = k % _NP
    cw = wids % _NC
    sw = wids // _NC
    qi = cw * _NP + pw
    qs = qb[qi]
    ql = qb[qi + 1] - qb[qi]
    start_w = qs + (sw * ql) // _NS
    end_w = qs + ((sw + 1) * ql) // _NS
    z7 = jnp.zeros((nq, 7), jnp.int32)
    table = jnp.concatenate(
        [start_w[:, None], z7, end_w[:, None], z7], axis=1
    ).reshape(nq * 16)
    table = jnp.concatenate([table, jnp.zeros((32,), jnp.int32)])

    sdl = sdst % _QR
    scol = jnp.concatenate([scol, jnp.zeros((_EPAD - _E,), jnp.int32)])
    sval = jnp.concatenate([sval, jnp.zeros((_EPAD - _E,), jnp.float32)])
    sdl = jnp.concatenate([sdl, jnp.zeros((_EPAD - _E,), jnp.int32)])
    idx2d = sdl.reshape(_EPAD // _CHUNK, _CHUNK)

    z0p = jnp.pad(Z0, ((0, _NPAD - _N), (0, 0)))
    z0ap = jnp.pad(Z0a, ((0, _NPAD - _N), (0, 0)))
    z = z0p
    for _ in range(_HOPS):
        z = _hop(z, z0ap, scol, sval, idx2d, table)
    return z[:_N]


# speedup vs baseline: 1.1863x; 1.1863x over previous
import functools

import jax
import jax.numpy as jnp
from jax import lax
from jax.experimental import pallas as pl
from jax.experimental.pallas import tpu as pltpu
from jax.experimental.pallas import tpu_sc as plsc

_N = 10000
_E = 160000
_IN = 256
_HID = 512
_OUT = 256
_D = _OUT
_HOPS = 10
_ALPHA = 0.1

_NC = 2
_NS = 16
_L = 16
_NW = _NC * _NS
_NP = 2
_QR = 2560
_RPT = _QR // _NS
_NPAD = _NC * _NP * _QR
_CAPE = 8192
_CHUNK = 32
_EPAD = _E + _CAPE


def _mlp_body(x_ref, w1_ref, b1_ref, w2_ref, b2_ref, o_ref, oa_ref):
    h = jnp.dot(x_ref[...], w1_ref[...], preferred_element_type=jnp.float32)
    h = jnp.maximum(h + b1_ref[...], 0.0)
    o = jnp.dot(h, w2_ref[...], preferred_element_type=jnp.float32)
    o = o + b2_ref[...]
    o_ref[...] = o
    oa_ref[...] = o * _ALPHA


def _mlp(X, W1, b1, W2, b2):
    BR = 1000
    return pl.pallas_call(
        _mlp_body,
        grid=(_N // BR,),
        in_specs=[
            pl.BlockSpec((BR, _IN), lambda i: (i, 0)),
            pl.BlockSpec((_IN, _HID), lambda i: (0, 0)),
            pl.BlockSpec((1, _HID), lambda i: (0, 0)),
            pl.BlockSpec((_HID, _OUT), lambda i: (0, 0)),
            pl.BlockSpec((1, _OUT), lambda i: (0, 0)),
        ],
        out_specs=[
            pl.BlockSpec((BR, _OUT), lambda i: (i, 0)),
            pl.BlockSpec((BR, _OUT), lambda i: (i, 0)),
        ],
        out_shape=[
            jax.ShapeDtypeStruct((_N, _OUT), jnp.float32),
            jax.ShapeDtypeStruct((_N, _OUT), jnp.float32),
        ],
    )(X, W1, b1.reshape(1, _HID), W2, b2.reshape(1, _OUT))


def _splat(v, j):
    idx = jnp.full((_L, 1), j, jnp.int32)
    dn = lax.GatherDimensionNumbers(
        offset_dims=(), collapsed_slice_dims=(0,), start_index_map=(0,))
    return lax.gather(v, idx, dn, slice_sizes=(1,),
                      mode=lax.GatherScatterMode.PROMISE_IN_BOUNDS)


def _make_hop():
    mesh = plsc.VectorSubcoreMesh(
        core_axis_name="c", subcore_axis_name="s",
        num_cores=_NC, num_subcores=_NS)

    @functools.partial(
        pl.kernel,
        out_type=jax.ShapeDtypeStruct((_NPAD, _D), jnp.float32),
        mesh=mesh,
        scratch_types=[
            pltpu.VMEM((_CAPE,), jnp.int32),
            pltpu.VMEM((_CAPE,), jnp.float32),
            pltpu.VMEM((_CAPE // _CHUNK, _CHUNK), jnp.int32),
            pltpu.VMEM((_CHUNK, _D), jnp.float32),
            pltpu.VMEM((_CHUNK, _D), jnp.float32),
            pltpu.VMEM((_CHUNK, _D), jnp.float32),
            pltpu.VMEM((_CHUNK, _D), jnp.float32),
            pltpu.VMEM((1056,), jnp.int32),
            pltpu.VMEM_SHARED((_QR, _D), jnp.float32),
            pltpu.SemaphoreType.DMA,
            pltpu.SemaphoreType.DMA,
            pltpu.SemaphoreType.DMA,
            pltpu.SemaphoreType.DMA,
        ],
        compiler_params=pltpu.CompilerParams(
            use_tc_tiling_on_sc=False, needs_layout_passes=False),
    )
    def hop(z_hbm, z0a_hbm, col_hbm, val_hbm, idx_hbm, st_hbm, out_hbm,
            colseg, valseg, idxseg, rb0, rb1, sb0, sb1, stv, accs,
            g0, g1, t0, t1):
        cid = lax.axis_index("c")
        sid = lax.axis_index("s")
        wid = sid * _NC + cid
        srow = sid * _RPT
        iota = lax.iota(jnp.int32, _L)

        pltpu.sync_copy(st_hbm, stv)

        def _gather(ci, rb, sem):
            src = z_hbm.at[colseg.at[pl.ds(ci * _CHUNK, _CHUNK)]]
            pltpu.async_copy(src, rb, sem)

        def _gwait(ci, rb, sem):
            src = z_hbm.at[colseg.at[pl.ds(ci * _CHUNK, _CHUNK)]]
            pltpu.make_async_copy(src, rb, sem).wait()

        def _scatter(ci, sb, sem):
            pltpu.async_copy(sb, accs.at[idxseg.at[ci]], sem, add=True)

        def _twait(sb, sem):
            pltpu.make_async_copy(sb, accs.at[idxseg.at[0]], sem).wait()

        def _scale(cs, ci, rb, sb, start, end):
            ebase = ci * _CHUNK
            nk = _D // _L
            lat = 4
            for g in range(_CHUNK // _L):
                off = ebase + g * _L
                eidx = (cs + off) + iota
                valid = (eidx >= start) & (eidx < end)
                valv = valseg[pl.ds(off, _L)]
                a = jnp.where(valid, valv, 0.0)
                for j in range(_L):
                    av = _splat(a, j)
                    rref = rb.at[g * _L + j]
                    sref = sb.at[g * _L + j]
                    rv = [None] * nk
                    pv = [None] * nk
                    for t in range(nk + lat + 1):
                        if t < nk:
                            rv[t] = rref[pl.ds(t * _L, _L)]
                        if lat <= t < nk + lat:
                            k = t - lat
                            pv[k] = av * rv[k]
                        if lat + 1 <= t:
                            k = t - lat - 1
                            sref[pl.ds(k * _L, _L)] = pv[k]

        for p in range(_NP):
            tb = wid * (_NP * 16) + p * 16
            start = stv[pl.ds(pl.multiple_of(tb, 8), _L)][0]
            end = stv[pl.ds(pl.multiple_of(tb + 8, 8), _L)][0]
            grow = (cid * _NP + p) * _QR + sid * _RPT

            pltpu.sync_copy(z0a_hbm.at[pl.ds(grow, _RPT)],
                            accs.at[pl.ds(srow, _RPT)])
            plsc.subcore_barrier()

            astart = start & (-256)
            total = end - astart
            nseg = lax.div(total + (_CAPE - 1), _CAPE)

            @pl.loop(0, nseg)
            def _seg(si):
                soff = pl.multiple_of(astart + si * _CAPE, 256)
                pltpu.sync_copy(col_hbm.at[pl.ds(soff, _CAPE)], colseg)
                pltpu.sync_copy(val_hbm.at[pl.ds(soff, _CAPE)], valseg)
                pltpu.sync_copy(
                    idx_hbm.at[pl.ds(lax.div(soff, _CHUNK), _CAPE // _CHUNK)],
                    idxseg)
                seg_n = jnp.minimum(end - soff, _CAPE)
                nch = lax.div(seg_n + (_CHUNK - 1), _CHUNK)
                nch2 = lax.div(nch + 1, 2) * 2

                _gather(0, rb0, g0)
                _gather(1, rb1, g1)

                @pl.loop(0, nch2, step=2)
                def _c(ci):
                    _gwait(ci, rb0, g0)

                    @pl.when(ci > 0)
                    def _():
                        _twait(sb0, t0)

                    _scale(soff, ci, rb0, sb0, start, end)
                    _scatter(ci, sb0, t0)

                    @pl.when(ci + 2 < nch2)
                    def _():
                        _gather(ci + 2, rb0, g0)

                    _gwait(ci + 1, rb1, g1)

                    @pl.when(ci > 0)
                    def _():
                        _twait(sb1, t1)

                    _scale(soff, ci + 1, rb1, sb1, start, end)
                    _scatter(ci + 1, sb1, t1)

                    @pl.when(ci + 3 < nch2)
                    def _():
                        _gather(ci + 3, rb1, g1)

                _twait(sb0, t0)
                _twait(sb1, t1)

            plsc.subcore_barrier()
            pltpu.sync_copy(accs.at[pl.ds(srow, _RPT)],
                            out_hbm.at[pl.ds(grow, _RPT)])

    return hop


_hop = _make_hop()


def kernel(X, edge_index, A_val, W1, b1, W2, b2):
    Z0, Z0a = _mlp(X, W1, b1, W2, b2)

    row = edge_index[0]
    col = edge_index[1]
    order = jnp.argsort(row)
    sdst = row[order]
    scol = col[order]
    sval = A_val[order] * (1.0 - _ALPHA)

    qb = jnp.searchsorted(
        sdst, jnp.arange(_NC * _NP + 1, dtype=jnp.int32) * _QR, side="left"
    ).astype(jnp.int32)
    nq = _NW * _NP
    k = jnp.arange(nq, dtype=jnp.int32)
    wids = k // _NP
    pw = k % _NP
    cw = wids % _NC
    sw = wids // _NC
    qi = cw * _NP + pw
    qs = qb[qi]
    ql = qb[qi + 1] - qb[qi]
    start_w = qs + (sw * ql) // _NS
    end_w = qs + ((sw + 1) * ql) // _NS
    z7 = jnp.zeros((nq, 7), jnp.int32)
    table = jnp.concatenate(
        [start_w[:, None], z7, end_w[:, None], z7], axis=1
    ).reshape(nq * 16)
    table = jnp.concatenate([table, jnp.zeros((32,), jnp.int32)])

    sdl = sdst % _QR
    scol = jnp.concatenate([scol, jnp.zeros((_EPAD - _E,), jnp.int32)])
    sval = jnp.concatenate([sval, jnp.zeros((_EPAD - _E,), jnp.float32)])
    sdl = jnp.concatenate([sdl, jnp.zeros((_EPAD - _E,), jnp.int32)])
    idx2d = sdl.reshape(_EPAD // _CHUNK, _CHUNK)

    z0p = jnp.pad(Z0, ((0, _NPAD - _N), (0, 0)))
    z0ap = jnp.pad(Z0a, ((0, _NPAD - _N), (0, 0)))
    z = z0p
    for _ in range(_HOPS):
        z = _hop(z, z0ap, scol, sval, idx2d, table)
    return z[:_N]
